# baseline (speedup 1.0000x reference)
import jax
import jax.numpy as jnp
from jax.experimental import pallas as pl
from jax.experimental.pallas import tpu as pltpu

VOCAB = 32000
D = 1024
MAXP = 16
TEMP = 1.0
NB = 3200
B = 128
BB = 32
K = 250
J = 128
KP = 256


def _h_kernel(a_ref, wh_ref, o_ref):
    o_ref[...] = jnp.tanh(jnp.dot(a_ref[...], wh_ref[...],
                                  preferred_element_type=jnp.float32))


def _logits_kernel(h_ref, w_ref, o_ref):
    o_ref[...] = jnp.dot(h_ref[...], w_ref[...],
                         preferred_element_type=jnp.float32)


def _pallas_h(a, W_h):
    return pl.pallas_call(
        _h_kernel,
        out_shape=jax.ShapeDtypeStruct((B, D), jnp.float32),
    )(a, W_h)


def _pallas_logits(h, W_out):
    return pl.pallas_call(
        _logits_kernel,
        grid=(VOCAB // NB,),
        in_specs=[pl.BlockSpec((B, D), lambda i: (0, 0)),
                  pl.BlockSpec((D, NB), lambda i: (0, i))],
        out_specs=pl.BlockSpec((B, NB), lambda i: (0, i)),
        out_shape=jax.ShapeDtypeStruct((B, VOCAB), jnp.float32),
    )(h, W_out)


def _sample_kernel(p_ref, u_ref, tok_ref, q_ref, pt_ref, tot_ref):
    x = p_ref[...].reshape(BB, K, J)
    for j0 in range(0, J, 16):
        sl = x[:, :, j0:j0 + 16]
        sl = jnp.swapaxes(sl, 1, 2)
        sl = jnp.swapaxes(sl, 0, 1)
        q_ref[j0:j0 + 16, :, :K] = sl
    q_ref[:, :, K:] = jnp.zeros((J, BB, KP - K), jnp.float32)

    def scan_body(j, _):
        q_ref[pl.ds(j, 1)] = q_ref[pl.ds(j, 1)] + q_ref[pl.ds(j - 1, 1)]
        return 0
    jax.lax.fori_loop(1, J, scan_body, 0, unroll=False)

    tot_ref[...] = q_ref[J - 1]

    carry = jnp.zeros((BB, 1), jnp.float32)
    for k in range(128):
        pt_ref[:, k:k + 1] = carry
        carry = carry + tot_ref[:, k:k + 1]
    base = carry
    S = jnp.zeros((BB, 1), jnp.float32)
    for k in range(128, K):
        pt_ref[:, k:k + 1] = S + base
        S = S + tot_ref[:, k:k + 1]
    pt_ref[:, K:] = jnp.full((BB, KP - K), jnp.inf, jnp.float32)

    u = u_ref[...]
    P = pt_ref[...]

    def cbody(j, acc):
        cs = q_ref[pl.ds(j, 1)][0] + P
        return acc + (cs < u).astype(jnp.int32)
    acc = jax.lax.fori_loop(0, J, cbody, jnp.zeros((BB, KP), jnp.int32),
                            unroll=False)
    cnt = jnp.sum(acc, axis=1, keepdims=True)
    tok_ref[...] = jnp.clip(cnt, 0, VOCAB - 1)


def _pallas_sample(p, u):
    return pl.pallas_call(
        _sample_kernel,
        grid=(B // BB,),
        in_specs=[pl.BlockSpec((BB, VOCAB), lambda i: (i, 0)),
                  pl.BlockSpec((BB, 1), lambda i: (i, 0))],
        out_specs=pl.BlockSpec((BB, 1), lambda i: (i, 0)),
        out_shape=jax.ShapeDtypeStruct((B, 1), jnp.int32),
        scratch_shapes=[pltpu.VMEM((J, BB, KP), jnp.float32),
                        pltpu.VMEM((BB, KP), jnp.float32),
                        pltpu.VMEM((BB, KP), jnp.float32)],
    )(p, u)


def kernel(X, E, W_h, W_out, rand_u):
    Bn = X.shape[0]
    ctx = jnp.mean(jnp.take(E, X, axis=0), axis=1)
    Y = jnp.ones((Bn, 1), dtype=jnp.int32)
    log_probabilities = jnp.zeros((Bn,), dtype=jnp.float32)
    for i in range(MAXP):
        a = jnp.take(E, Y[:, -1], axis=0) + ctx
        h = _pallas_h(a, W_h)
        next_log_probabilities = _pallas_logits(h, W_out)
        next_probabilities = jax.nn.softmax(next_log_probabilities / TEMP, axis=1)
        next_chars = _pallas_sample(next_probabilities, rand_u[i])
        lp = jax.nn.log_softmax(next_log_probabilities / TEMP, axis=1)
        log_probabilities = log_probabilities + jnp.take_along_axis(lp, next_chars, axis=1)[:, 0]
        Y = jnp.concatenate([Y, next_chars], axis=1)
    return Y, log_probabilities


# speedup vs baseline: 1.5503x; 1.5503x over previous
import jax
import jax.numpy as jnp
from jax.experimental import pallas as pl
from jax.experimental.pallas import tpu as pltpu

VOCAB = 32000
D = 1024
MAXP = 16
TEMP = 1.0
NB = 3200
B = 128
K = 250
J = 128
KP = 256


def _h_kernel(a_ref, wh_ref, o_ref):
    o_ref[...] = jnp.tanh(jnp.dot(a_ref[...], wh_ref[...],
                                  preferred_element_type=jnp.float32))


def _logits_kernel(h_ref, w_ref, o_ref):
    o_ref[...] = jnp.dot(h_ref[...], w_ref[...],
                         preferred_element_type=jnp.float32)


def _pallas_h(a, W_h):
    return pl.pallas_call(
        _h_kernel,
        out_shape=jax.ShapeDtypeStruct((B, D), jnp.float32),
    )(a, W_h)


def _pallas_logits(h, W_out):
    return pl.pallas_call(
        _logits_kernel,
        grid=(VOCAB // NB,),
        in_specs=[pl.BlockSpec((B, D), lambda i: (0, 0)),
                  pl.BlockSpec((D, NB), lambda i: (0, i))],
        out_specs=pl.BlockSpec((B, NB), lambda i: (0, i)),
        out_shape=jax.ShapeDtypeStruct((B, VOCAB), jnp.float32),
    )(h, W_out)


def _sample_kernel(t_ref, u_ref, tok_ref, q_ref, pt_ref, tot_ref):
    q_ref[...] = t_ref[...]

    def scan_body(j, _):
        q_ref[pl.ds(j, 1)] = q_ref[pl.ds(j, 1)] + q_ref[pl.ds(j - 1, 1)]
        return 0
    jax.lax.fori_loop(1, J, scan_body, 0, unroll=False)

    tot_ref[...] = q_ref[J - 1]

    carry = jnp.zeros((B, 1), jnp.float32)
    for k in range(128):
        pt_ref[:, k:k + 1] = carry
        carry = carry + tot_ref[:, k:k + 1]
    base = carry
    S = jnp.zeros((B, 1), jnp.float32)
    for k in range(128, K):
        pt_ref[:, k:k + 1] = S + base
        S = S + tot_ref[:, k:k + 1]
    pt_ref[:, K:] = jnp.full((B, KP - K), jnp.inf, jnp.float32)

    u = u_ref[...]
    P = pt_ref[...]

    def cbody(j, acc):
        cs = q_ref[pl.ds(j, 1)][0] + P
        return acc + (cs < u).astype(jnp.int32)
    acc = jax.lax.fori_loop(0, J, cbody, jnp.zeros((B, KP), jnp.int32),
                            unroll=False)
    cnt = jnp.sum(acc, axis=1, keepdims=True)
    tok_ref[...] = jnp.clip(cnt, 0, VOCAB - 1)


def _pallas_sample(p, u):
    t = jnp.transpose(p.reshape(B, K, J), (2, 0, 1))
    t = jnp.pad(t, ((0, 0), (0, 0), (0, KP - K)))
    return pl.pallas_call(
        _sample_kernel,
        out_shape=jax.ShapeDtypeStruct((B, 1), jnp.int32),
        scratch_shapes=[pltpu.VMEM((J, B, KP), jnp.float32),
                        pltpu.VMEM((B, KP), jnp.float32),
                        pltpu.VMEM((B, KP), jnp.float32)],
    )(t, u)


def kernel(X, E, W_h, W_out, rand_u):
    Bn = X.shape[0]
    ctx = jnp.mean(jnp.take(E, X, axis=0), axis=1)
    Y = jnp.ones((Bn, 1), dtype=jnp.int32)
    log_probabilities = jnp.zeros((Bn,), dtype=jnp.float32)
    for i in range(MAXP):
        a = jnp.take(E, Y[:, -1], axis=0) + ctx
        h = _pallas_h(a, W_h)
        x = _pallas_logits(h, W_out) / TEMP
        next_probabilities = jax.nn.softmax(x, axis=1)
        next_chars = _pallas_sample(next_probabilities, rand_u[i])
        m = jax.lax.stop_gradient(jnp.max(x, axis=1, keepdims=True))
        s = jnp.sum(jnp.exp(x - m), axis=1, keepdims=True)
        lp_val = (jnp.take_along_axis(x, next_chars, axis=1) - m) - jnp.log(s)
        log_probabilities = log_probabilities + lp_val[:, 0]
        Y = jnp.concatenate([Y, next_chars], axis=1)
    return Y, log_probabilities
